# baseline (speedup 1.0000x reference)
import functools

import jax
import jax.numpy as jnp
from jax import lax
from jax.experimental import pallas as pl
from jax.experimental.pallas import tpu as pltpu
from jax.experimental.pallas import tpu_sc as plsc

SEQ_LEN = 8192
MODEL_DIM = 1024

_NC = 2
_NS = 16
_NW = _NC * _NS
_ROWS_PER_W = SEQ_LEN // _NW

_mesh = plsc.VectorSubcoreMesh(core_axis_name="c", subcore_axis_name="s")


@functools.partial(
    pl.kernel,
    mesh=_mesh,
    out_type=jax.ShapeDtypeStruct((SEQ_LEN, MODEL_DIM), jnp.float32),
)
def _copy_rows(table_hbm, out_hbm):
    wid = lax.axis_index("s") * _NC + lax.axis_index("c")
    base = wid * _ROWS_PER_W
    pltpu.sync_copy(
        table_hbm.at[pl.ds(base, _ROWS_PER_W)],
        out_hbm.at[pl.ds(base, _ROWS_PER_W)],
    )


def kernel(x, emb_weight):
    del x
    return _copy_rows(emb_weight)


# speedup vs baseline: 24.9973x; 24.9973x over previous
import functools

import jax
import jax.numpy as jnp
from jax import lax
from jax.experimental import pallas as pl
from jax.experimental.pallas import tpu as pltpu
from jax.experimental.pallas import tpu_sc as plsc

SEQ_LEN = 8192
MODEL_DIM = 1024

_NC = 2
_NS = 16
_NW = _NC * _NS
_ROWS_PER_W = SEQ_LEN // _NW
_CHUNK = 16
_NSTEPS = _ROWS_PER_W // _CHUNK
_NBUF = 7

_mesh = plsc.VectorSubcoreMesh(core_axis_name="c", subcore_axis_name="s")


@functools.partial(
    pl.kernel,
    mesh=_mesh,
    out_type=jax.ShapeDtypeStruct((SEQ_LEN, MODEL_DIM), jnp.float32),
    scratch_types=[
        pltpu.VMEM((_NBUF, _CHUNK, MODEL_DIM), jnp.float32),
        pltpu.SemaphoreType.DMA((_NBUF,)),
        pltpu.SemaphoreType.DMA((_NBUF,)),
    ],
)
def _copy_rows(table_hbm, out_hbm, buf, sem_in, sem_out):
    wid = lax.axis_index("s") * _NC + lax.axis_index("c")
    base = wid * _ROWS_PER_W

    in_cp = [None] * _NSTEPS
    out_cp = [None] * _NSTEPS

    def start_in(step):
        b = step % _NBUF
        return pltpu.async_copy(
            table_hbm.at[pl.ds(base + step * _CHUNK, _CHUNK)],
            buf.at[b],
            sem_in.at[b],
        )

    for step in range(min(_NBUF, _NSTEPS)):
        in_cp[step] = start_in(step)

    for step in range(_NSTEPS):
        b = step % _NBUF
        in_cp[step].wait()
        out_cp[step] = pltpu.async_copy(
            buf.at[b],
            out_hbm.at[pl.ds(base + step * _CHUNK, _CHUNK)],
            sem_out.at[b],
        )
        prev = step - 1
        nxt = prev + _NBUF
        if prev >= 0 and nxt < _NSTEPS:
            out_cp[prev].wait()
            in_cp[nxt] = start_in(nxt)

    for step in range(max(0, _NSTEPS - _NBUF), _NSTEPS):
        out_cp[step].wait()


def kernel(x, emb_weight):
    del x
    return _copy_rows(emb_weight)
